# Initial kernel scaffold; baseline (speedup 1.0000x reference)
import jax
import jax.numpy as jnp
from jax.experimental import pallas as pl


def kernel(x, edge_index, edge_probs, W1, b1, W2, b2):
    raise NotImplementedError("write your pallas kernel here")


# speedup vs baseline: 7.1762x; 7.1762x over previous
import functools

import jax
import jax.numpy as jnp
from jax import lax
from jax.experimental import pallas as pl
from jax.experimental.pallas import tpu as pltpu
from jax.experimental.pallas import tpu_sc as plsc

N = 10000
E = 320000
F_IN = 128
HID = 16
C = 16

NC = 2
NS = 16
NW = NC * NS
L = 16

NPAD = 10016
EPW = E // NW
GPW = EPW // L
CHUNK = 128
NCHUNK = 79
EPAD = NCHUNK * CHUNK
ROWS_PT = NPAD // NS

_mesh = plsc.VectorSubcoreMesh(core_axis_name="c", subcore_axis_name="s")


@functools.partial(
    pl.kernel,
    out_type=(
        jax.ShapeDtypeStruct((NW, NCHUNK, CHUNK), jnp.int32),
        jax.ShapeDtypeStruct((NW, NPAD), jnp.float32),
    ),
    mesh=_mesh,
    scratch_types=[
        pltpu.VMEM((EPW,), jnp.int32),
        pltpu.VMEM((EPW,), jnp.int32),
        pltpu.VMEM((EPW,), jnp.float32),
        pltpu.VMEM((EPW,), jnp.float32),
        pltpu.VMEM((NCHUNK, CHUNK), jnp.int32),
        pltpu.VMEM((NPAD,), jnp.float32),
    ],
    compiler_params=pltpu.CompilerParams(
        needs_layout_passes=False, use_tc_tiling_on_sc=False),
)
def _sc_prep(src_hbm, dst_hbm, ep_hbm, gum_hbm, sel_hbm, deg_hbm,
             srcv, dstv, epv, gumv, selv, degv):
    wid = lax.axis_index("s") * NC + lax.axis_index("c")
    pltpu.sync_copy(src_hbm.at[wid], srcv)
    pltpu.sync_copy(dst_hbm.at[wid], dstv)
    pltpu.sync_copy(ep_hbm.at[wid], epv)
    pltpu.sync_copy(gum_hbm.at[wid], gumv)

    def zero(i, carry):
        degv[pl.ds(i * L, L)] = jnp.zeros((L,), jnp.float32)
        return carry
    lax.fori_loop(0, NPAD // L, zero, 0)

    ones = jnp.ones((L,), jnp.float32)
    dummy = jnp.full((L,), N, jnp.int32)

    def body(g, carry):
        sl = pl.ds(g * L, L)
        y = epv[sl] + gumv[sl]
        m = y > 0.0
        sel = jnp.where(m, srcv[sl], dummy)
        selv[g // 8, pl.ds((g % 8) * L, L)] = sel
        plsc.addupdate_scatter(degv, [dstv[sl]], ones, mask=m)
        return carry
    lax.fori_loop(0, GPW, body, 0)

    def pad(g, carry):
        selv[g // 8, pl.ds((g % 8) * L, L)] = dummy
        return carry
    lax.fori_loop(GPW, EPAD // L, pad, 0)

    pltpu.sync_copy(selv, sel_hbm.at[wid])
    pltpu.sync_copy(degv, deg_hbm.at[wid])


@functools.partial(
    pl.kernel,
    out_type=jax.ShapeDtypeStruct((NC, NPAD, HID), jnp.float32),
    mesh=_mesh,
    scratch_types=[
        pltpu.VMEM((NCHUNK, CHUNK), jnp.int32),
        pltpu.VMEM((NCHUNK, CHUNK), jnp.int32),
        pltpu.VMEM((CHUNK, HID), jnp.float32),
        pltpu.VMEM((ROWS_PT, HID), jnp.float32),
        pltpu.VMEM_SHARED((NPAD, HID), jnp.float32),
        pltpu.SemaphoreType.DMA,
    ],
    compiler_params=pltpu.CompilerParams(
        needs_layout_passes=False, use_tc_tiling_on_sc=False),
)
def _sc_msg(g_hbm, sel_hbm, dstp_hbm, out_hbm, selv, dstv, buf, zbuf, acc, sem):
    c = lax.axis_index("c")
    s = lax.axis_index("s")
    wid = s * NC + c

    def zero(i, carry):
        zbuf[i, :] = jnp.zeros((L,), jnp.float32)
        return carry
    lax.fori_loop(0, ROWS_PT, zero, 0)
    pltpu.sync_copy(zbuf, acc.at[pl.ds(s * ROWS_PT, ROWS_PT)])
    plsc.subcore_barrier()

    pltpu.sync_copy(sel_hbm.at[wid], selv)
    pltpu.sync_copy(dstp_hbm.at[wid], dstv)

    def chunk(j, carry):
        pltpu.async_copy(g_hbm.at[selv.at[j]], buf, sem).wait()
        pltpu.sync_copy(buf, acc.at[dstv.at[j]], add=True)
        return carry
    lax.fori_loop(0, NCHUNK, chunk, 0)

    plsc.subcore_barrier()
    pltpu.sync_copy(acc.at[pl.ds(s * ROWS_PT, ROWS_PT)],
                    out_hbm.at[c, pl.ds(s * ROWS_PT, ROWS_PT)])


def _dinv_body(deg_ref, out_ref):
    d = jnp.sum(deg_ref[...], axis=1, keepdims=True) + 1.0
    out_ref[...] = lax.rsqrt(d)


def _g1_body(x_ref, w1_ref, dinv_ref, out_ref):
    h = jnp.dot(x_ref[...], w1_ref[...], preferred_element_type=jnp.float32)
    out_ref[...] = h * dinv_ref[...]


def _comb1_body(s_ref, g_ref, dinv_ref, b1_ref, w2_ref, out_ref):
    o = (s_ref[0] + s_ref[1] + g_ref[...]) * dinv_ref[...] + b1_ref[...]
    h = jnp.maximum(o, 0.0)
    g2 = jnp.dot(h, w2_ref[...], preferred_element_type=jnp.float32)
    g2 = g2 * dinv_ref[...]
    rows = lax.broadcasted_iota(jnp.int32, (NPAD, C), 0)
    out_ref[...] = jnp.where(rows < N, g2, 0.0)


def _comb2_body(s_ref, g_ref, dinv_ref, b2_ref, out_ref):
    o = (s_ref[0, :N] + s_ref[1, :N] + g_ref[:N]) * dinv_ref[:N] + b2_ref[...]
    m = jnp.max(o, axis=1, keepdims=True)
    lse = jnp.log(jnp.sum(jnp.exp(o - m), axis=1, keepdims=True)) + m
    out_ref[...] = o - lse


def kernel(x, edge_index, edge_probs, W1, b1, W2, b2):
    gkey = jax.random.key(42)
    u = jax.random.uniform(gkey, (E,), minval=1e-9, maxval=1.0 - 1e-9)
    gum = -jnp.log(-jnp.log(u))

    src = edge_index[0].reshape(NW, EPW)
    dst = edge_index[1].reshape(NW, EPW)
    ep2 = edge_probs.reshape(NW, EPW)
    gum2 = gum.reshape(NW, EPW)

    sel, deg_parts = _sc_prep(src, dst, ep2, gum2)

    dstp = jnp.concatenate(
        [dst, jnp.full((NW, EPAD - EPW), N, jnp.int32)], axis=1
    ).reshape(NW, NCHUNK, CHUNK)

    dinv = pl.pallas_call(
        _dinv_body,
        out_shape=jax.ShapeDtypeStruct((NPAD, 1), jnp.float32),
    )(deg_parts.T)

    xp = jnp.pad(x, ((0, NPAD - N), (0, 0)))
    g1 = pl.pallas_call(
        _g1_body,
        out_shape=jax.ShapeDtypeStruct((NPAD, HID), jnp.float32),
    )(xp, W1, dinv)

    s1 = _sc_msg(g1, sel, dstp)

    g2 = pl.pallas_call(
        _comb1_body,
        out_shape=jax.ShapeDtypeStruct((NPAD, C), jnp.float32),
    )(s1, g1, dinv, b1, W2)

    s2 = _sc_msg(g2, sel, dstp)

    out = pl.pallas_call(
        _comb2_body,
        out_shape=jax.ShapeDtypeStruct((N, C), jnp.float32),
    )(s2, g2, dinv, b2)
    return out
